# baseline (speedup 1.0000x reference)
import functools

import jax
import jax.numpy as jnp
from jax import lax
from jax.experimental import pallas as pl
from jax.experimental.pallas import tpu as pltpu
from jax.experimental.pallas import tpu_sc as plsc

_N = 10000
_E = 160000
_F = 128
_H1 = 32
_H2 = 16

_CHUNK = 128
_NCHUNKS = _E // _CHUNK
_NC = 2
_NS = 16
_NW = _NC * _NS
_CPW = (_NCHUNKS + _NW - 1) // _NW
_NCH_PAD = (_CPW + 2) * _NW
_RPT8 = (_N // _NS) // 8 * 8
_TAIL = _N - _NS * _RPT8

_ROWS_BLK = 1000
_DEC_BM = 512
_DEC_BN = _N


def _spmm_partials(table, edata, ew3, zeros):
    mesh = plsc.VectorSubcoreMesh(core_axis_name="c", subcore_axis_name="s")

    @functools.partial(
        pl.kernel,
        mesh=mesh,
        compiler_params=pltpu.CompilerParams(use_tc_tiling_on_sc=False),
        out_type=jax.ShapeDtypeStruct((_NC, _N, _H1), jnp.float32),
        scratch_types=[
            pltpu.VMEM((2, _CHUNK), jnp.int32),
            pltpu.VMEM((2, _CHUNK), jnp.int32),
            pltpu.VMEM((1, _CHUNK), jnp.float32),
            pltpu.VMEM((1, _CHUNK), jnp.float32),
            pltpu.VMEM((_CHUNK, _H1), jnp.float32),
            pltpu.VMEM((_CHUNK, _H1), jnp.float32),
            pltpu.VMEM_SHARED((_N, _H1), jnp.float32),
            pltpu.SemaphoreType.DMA,
            pltpu.SemaphoreType.DMA,
            pltpu.SemaphoreType.DMA,
            pltpu.SemaphoreType.DMA,
            pltpu.SemaphoreType.DMA,
            pltpu.SemaphoreType.DMA,
        ],
    )
    def k(table_hbm, edata_hbm, ew_hbm, zeros_hbm, out_hbm,
          ebuf0, ebuf1, wbuf0, wbuf1, rows0, rows1, acc,
          sem_e0, sem_e1, sem_w0, sem_w1, sem_g0, sem_g1):
        c = lax.axis_index("c")
        s = lax.axis_index("s")
        wid = s * _NC + c
        ebufs = (ebuf0, ebuf1)
        wbufs = (wbuf0, wbuf1)
        rowss = (rows0, rows1)
        sem_e = (sem_e0, sem_e1)
        sem_w = (sem_w0, sem_w1)
        sem_g = (sem_g0, sem_g1)

        def e_copies(g, b):
            chunk = g * _NW + wid
            return (
                pltpu.make_async_copy(edata_hbm.at[chunk], ebufs[b],
                                      sem_e[b]),
                pltpu.make_async_copy(ew_hbm.at[chunk], wbufs[b],
                                      sem_w[b]),
            )

        def g_copy(b):
            return pltpu.make_async_copy(table_hbm.at[ebufs[b].at[0]],
                                         rowss[b], sem_g[b])

        def scale(b):
            rows = rowss[b]
            for gi in range(_CHUNK // 16):
                wg = wbufs[b][0, pl.ds(gi * 16, 16)]
                for t in range(16):
                    e = gi * 16 + t
                    wb = lax.gather(
                        wg, jnp.full((16, 1), t, jnp.int32),
                        lax.GatherDimensionNumbers(
                            offset_dims=(), collapsed_slice_dims=(0,),
                            start_index_map=(0,)),
                        slice_sizes=(1,),
                        mode=lax.GatherScatterMode.PROMISE_IN_BOUNDS)
                    rows[e, pl.ds(0, 16)] = rows[e, pl.ds(0, 16)] * wb
                    rows[e, pl.ds(16, 16)] = rows[e, pl.ds(16, 16)] * wb

        for ac in e_copies(0, 0) + e_copies(1, 1):
            ac.start()
        pltpu.sync_copy(zeros_hbm.at[pl.ds(s * _RPT8, _RPT8)],
                        acc.at[pl.ds(s * _RPT8, _RPT8)])

        @pl.when(s == 0)
        def _():
            pltpu.sync_copy(zeros_hbm.at[pl.ds(_NS * _RPT8, _TAIL)],
                            acc.at[pl.ds(_NS * _RPT8, _TAIL)])

        plsc.subcore_barrier()
        for ac in e_copies(0, 0):
            ac.wait()
        g_copy(0).start()

        def pair_body(i, carry):
            for b in (0, 1):
                g = 2 * i + b
                b1 = 1 - b
                g_copy(b).wait()
                for ac in e_copies(g + 1, b1):
                    ac.wait()
                g_copy(b1).start()
                scale(b)
                pltpu.sync_copy(rowss[b], acc.at[ebufs[b].at[1]], add=True)
                for ac in e_copies(g + 2, b):
                    ac.start()
            return carry

        lax.fori_loop(0, _CPW // 2, pair_body, 0)

        g_copy(0).wait()
        for ac in e_copies(_CPW + 1, 1):
            ac.wait()

        plsc.subcore_barrier()
        pltpu.sync_copy(acc.at[pl.ds(s * _RPT8, _RPT8)],
                        out_hbm.at[c, pl.ds(s * _RPT8, _RPT8)])

        @pl.when(s == 0)
        def _():
            pltpu.sync_copy(acc.at[pl.ds(_NS * _RPT8, _TAIL)],
                            out_hbm.at[c, pl.ds(_NS * _RPT8, _TAIL)])

    return k(table, edata, ew3, zeros)


def _matmul_xw0(x, W0):
    def body(x_ref, w_ref, o_ref):
        o_ref[...] = jnp.dot(x_ref[...], w_ref[...],
                             preferred_element_type=jnp.float32)

    return pl.pallas_call(
        body,
        grid=(_N // _ROWS_BLK,),
        in_specs=[
            pl.BlockSpec((_ROWS_BLK, _F), lambda i: (i, 0)),
            pl.BlockSpec((_F, _H1), lambda i: (0, 0)),
        ],
        out_specs=pl.BlockSpec((_ROWS_BLK, _H1), lambda i: (i, 0)),
        out_shape=jax.ShapeDtypeStruct((_N, _H1), jnp.float32),
    )(x, W0)


def _relu_combine(p):
    def body(p_ref, o_ref):
        o_ref[...] = jnp.maximum(p_ref[0] + p_ref[1], 0.0)

    return pl.pallas_call(
        body,
        grid=(_N // _ROWS_BLK,),
        in_specs=[pl.BlockSpec((_NC, _ROWS_BLK, _H1), lambda i: (0, i, 0))],
        out_specs=pl.BlockSpec((_ROWS_BLK, _H1), lambda i: (i, 0)),
        out_shape=jax.ShapeDtypeStruct((_N, _H1), jnp.float32),
    )(p)


def _z_combine(q, W1, W2, eps):
    def body(q_ref, w1_ref, w2_ref, e_ref, o_ref):
        sblk = q_ref[0] + q_ref[1]
        zm = jnp.dot(sblk, w1_ref[...], preferred_element_type=jnp.float32)
        zl = jnp.dot(sblk, w2_ref[...], preferred_element_type=jnp.float32)
        o_ref[...] = zm + e_ref[...] * jnp.exp(zl)

    return pl.pallas_call(
        body,
        grid=(_N // _ROWS_BLK,),
        in_specs=[
            pl.BlockSpec((_NC, _ROWS_BLK, _H1), lambda i: (0, i, 0)),
            pl.BlockSpec((_H1, _H2), lambda i: (0, 0)),
            pl.BlockSpec((_H1, _H2), lambda i: (0, 0)),
            pl.BlockSpec((_ROWS_BLK, _H2), lambda i: (i, 0)),
        ],
        out_specs=pl.BlockSpec((_ROWS_BLK, _H2), lambda i: (i, 0)),
        out_shape=jax.ShapeDtypeStruct((_N, _H2), jnp.float32),
    )(q, W1, W2, eps)


_DEC_P = 8
_DEC_PR = 128


def _decode_manual(zh, zl):
    n_steps = _N // 1000
    dn = (((1,), (1,)), ((), ()))

    def body(zih_ref, zil_ref, zjh_ref, zjl_ref, o_hbm, scratch, sems):
        i = pl.program_id(0)
        for p in range(_DEC_P):
            rows = _DEC_PR if p < _DEC_P - 1 else 1000 - (_DEC_P - 1) * _DEC_PR
            r0 = p * _DEC_PR

            def mk_copy(step):
                return pltpu.make_async_copy(
                    scratch.at[pl.ds(r0, rows)],
                    o_hbm.at[pl.ds(step * 1000 + r0, rows)],
                    sems.at[p],
                )

            @pl.when(i > 0)
            def _():
                mk_copy(i - 1).wait()

            zih = zih_ref[pl.ds(r0, rows), :]
            zil = zil_ref[pl.ds(r0, rows), :]
            scratch[pl.ds(r0, rows)] = (
                lax.dot_general(zih, zjh_ref[...], dn,
                                preferred_element_type=jnp.float32)
                + lax.dot_general(zih, zjl_ref[...], dn,
                                  preferred_element_type=jnp.float32)
                + lax.dot_general(zil, zjh_ref[...], dn,
                                  preferred_element_type=jnp.float32)
            )
            mk_copy(i).start()

        @pl.when(i == n_steps - 1)
        def _():
            for p in range(_DEC_P):
                rows = _DEC_PR if p < _DEC_P - 1 else 1000 - (_DEC_P - 1) * _DEC_PR
                r0 = p * _DEC_PR
                pltpu.make_async_copy(
                    scratch.at[pl.ds(r0, rows)],
                    o_hbm.at[pl.ds(i * 1000 + r0, rows)],
                    sems.at[p],
                ).wait()

    return pl.pallas_call(
        body,
        grid=(n_steps,),
        in_specs=[
            pl.BlockSpec((1000, _H2), lambda i: (i, 0)),
            pl.BlockSpec((1000, _H2), lambda i: (i, 0)),
            pl.BlockSpec((_N, _H2), lambda i: (0, 0)),
            pl.BlockSpec((_N, _H2), lambda i: (0, 0)),
        ],
        out_specs=pl.BlockSpec(memory_space=pl.ANY),
        out_shape=jax.ShapeDtypeStruct((_N, _N), jnp.float32),
        scratch_shapes=[
            pltpu.VMEM((1000, _N), jnp.float32),
            pltpu.SemaphoreType.DMA((_DEC_P,)),
        ],
    )(zh, zl, zh, zl)


def kernel(x, edge_index, edge_weight, W0, W1, W2):
    src = edge_index[0]
    dst = edge_index[1]
    edata = jnp.zeros((_NCH_PAD, 2, _CHUNK), jnp.int32).at[:_NCHUNKS].set(
        jnp.stack(
            [src.reshape(_NCHUNKS, _CHUNK),
             dst.reshape(_NCHUNKS, _CHUNK)],
            axis=1,
        ))
    ew3 = jnp.zeros((_NCH_PAD, 1, _CHUNK), jnp.float32).at[:_NCHUNKS].set(
        edge_weight.reshape(_NCHUNKS, 1, _CHUNK))
    zeros = jnp.zeros((_N, _H1), jnp.float32)

    h0 = _matmul_xw0(x, W0)
    p = _spmm_partials(h0, edata, ew3, zeros)
    hidden1 = _relu_combine(p)
    q = _spmm_partials(hidden1, edata, ew3, zeros)

    eps = jax.random.normal(jax.random.key(42), (_N, _H2), dtype=jnp.float32)
    z = _z_combine(q, W1, W2, eps)
    zh = z.astype(jnp.bfloat16)
    zl = (z - zh.astype(jnp.float32)).astype(jnp.bfloat16)
    return _decode_manual(zh, zl).reshape(-1)


# speedup vs baseline: 1.0648x; 1.0648x over previous
import functools

import jax
import jax.numpy as jnp
from jax import lax
from jax.experimental import pallas as pl
from jax.experimental.pallas import tpu as pltpu
from jax.experimental.pallas import tpu_sc as plsc

_N = 10000
_E = 160000
_F = 128
_H1 = 32
_H2 = 16

_CHUNK = 128
_NCHUNKS = _E // _CHUNK
_NC = 2
_NS = 16
_NW = _NC * _NS
_CPW = (_NCHUNKS + _NW - 1) // _NW
_NCH_PAD = (_CPW + 2) * _NW
_RPT8 = (_N // _NS) // 8 * 8
_TAIL = _N - _NS * _RPT8

_ROWS_BLK = 1000
_DEC_BM = 512
_DEC_BN = _N


def _spmm_partials(table, edata, ew3, zeros):
    mesh = plsc.VectorSubcoreMesh(core_axis_name="c", subcore_axis_name="s")

    @functools.partial(
        pl.kernel,
        mesh=mesh,
        compiler_params=pltpu.CompilerParams(use_tc_tiling_on_sc=False),
        out_type=jax.ShapeDtypeStruct((_NC, _N, _H1), jnp.float32),
        scratch_types=[
            pltpu.VMEM((2, _CHUNK), jnp.int32),
            pltpu.VMEM((1, _CHUNK), jnp.float32),
            pltpu.VMEM((_CHUNK, _H1), jnp.float32),
            pltpu.VMEM_SHARED((_N, _H1), jnp.float32),
            pltpu.SemaphoreType.DMA,
        ],
    )
    def k(table_hbm, edata_hbm, ew_hbm, zeros_hbm, out_hbm,
          ebuf, wbuf, rows, acc, sem):
        c = lax.axis_index("c")
        s = lax.axis_index("s")
        wid = s * _NC + c

        pltpu.sync_copy(zeros_hbm.at[pl.ds(s * _RPT8, _RPT8)],
                        acc.at[pl.ds(s * _RPT8, _RPT8)])

        @pl.when(s == 0)
        def _():
            pltpu.sync_copy(zeros_hbm.at[pl.ds(_NS * _RPT8, _TAIL)],
                            acc.at[pl.ds(_NS * _RPT8, _TAIL)])

        plsc.subcore_barrier()

        def chunk_body(i, carry):
            chunk = i * _NW + wid

            @pl.when(chunk < _NCHUNKS)
            def _():
                pltpu.sync_copy(edata_hbm.at[chunk], ebuf)
                pltpu.sync_copy(ew_hbm.at[chunk], wbuf)
                pltpu.async_copy(table_hbm.at[ebuf.at[0]], rows, sem).wait()
                for g in range(_CHUNK // 16):
                    wg = wbuf[0, pl.ds(g * 16, 16)]
                    for t in range(16):
                        e = g * 16 + t
                        wb = lax.gather(
                            wg, jnp.full((16, 1), t, jnp.int32),
                            lax.GatherDimensionNumbers(
                                offset_dims=(), collapsed_slice_dims=(0,),
                                start_index_map=(0,)),
                            slice_sizes=(1,),
                            mode=lax.GatherScatterMode.PROMISE_IN_BOUNDS)
                        rows[e, pl.ds(0, 16)] = rows[e, pl.ds(0, 16)] * wb
                        rows[e, pl.ds(16, 16)] = rows[e, pl.ds(16, 16)] * wb
                pltpu.sync_copy(rows, acc.at[ebuf.at[1]], add=True)

            return carry

        lax.fori_loop(0, _CPW, chunk_body, 0)
        plsc.subcore_barrier()
        pltpu.sync_copy(acc.at[pl.ds(s * _RPT8, _RPT8)],
                        out_hbm.at[c, pl.ds(s * _RPT8, _RPT8)])

        @pl.when(s == 0)
        def _():
            pltpu.sync_copy(acc.at[pl.ds(_NS * _RPT8, _TAIL)],
                            out_hbm.at[c, pl.ds(_NS * _RPT8, _TAIL)])

    return k(table, edata, ew3, zeros)


def _matmul_xw0(x, W0):
    def body(x_ref, w_ref, o_ref):
        o_ref[...] = jnp.dot(x_ref[...], w_ref[...],
                             preferred_element_type=jnp.float32)

    return pl.pallas_call(
        body,
        grid=(_N // _ROWS_BLK,),
        in_specs=[
            pl.BlockSpec((_ROWS_BLK, _F), lambda i: (i, 0)),
            pl.BlockSpec((_F, _H1), lambda i: (0, 0)),
        ],
        out_specs=pl.BlockSpec((_ROWS_BLK, _H1), lambda i: (i, 0)),
        out_shape=jax.ShapeDtypeStruct((_N, _H1), jnp.float32),
    )(x, W0)


def _relu_combine(p):
    def body(p_ref, o_ref):
        o_ref[...] = jnp.maximum(p_ref[0] + p_ref[1], 0.0)

    return pl.pallas_call(
        body,
        grid=(_N // _ROWS_BLK,),
        in_specs=[pl.BlockSpec((_NC, _ROWS_BLK, _H1), lambda i: (0, i, 0))],
        out_specs=pl.BlockSpec((_ROWS_BLK, _H1), lambda i: (i, 0)),
        out_shape=jax.ShapeDtypeStruct((_N, _H1), jnp.float32),
    )(p)


def _z_combine(q, W1, W2, eps):
    def body(q_ref, w1_ref, w2_ref, e_ref, o_ref):
        sblk = q_ref[0] + q_ref[1]
        zm = jnp.dot(sblk, w1_ref[...], preferred_element_type=jnp.float32)
        zl = jnp.dot(sblk, w2_ref[...], preferred_element_type=jnp.float32)
        o_ref[...] = zm + e_ref[...] * jnp.exp(zl)

    return pl.pallas_call(
        body,
        grid=(_N // _ROWS_BLK,),
        in_specs=[
            pl.BlockSpec((_NC, _ROWS_BLK, _H1), lambda i: (0, i, 0)),
            pl.BlockSpec((_H1, _H2), lambda i: (0, 0)),
            pl.BlockSpec((_H1, _H2), lambda i: (0, 0)),
            pl.BlockSpec((_ROWS_BLK, _H2), lambda i: (i, 0)),
        ],
        out_specs=pl.BlockSpec((_ROWS_BLK, _H2), lambda i: (i, 0)),
        out_shape=jax.ShapeDtypeStruct((_N, _H2), jnp.float32),
    )(q, W1, W2, eps)


_DEC_P = 8
_DEC_PR = 128


def _decode_manual(zh, zl):
    n_steps = _N // 1000
    dn = (((1,), (1,)), ((), ()))

    def body(zih_ref, zil_ref, zjh_ref, zjl_ref, o_hbm, scratch, sems):
        i = pl.program_id(0)
        for p in range(_DEC_P):
            rows = _DEC_PR if p < _DEC_P - 1 else 1000 - (_DEC_P - 1) * _DEC_PR
            r0 = p * _DEC_PR

            def mk_copy(step):
                return pltpu.make_async_copy(
                    scratch.at[pl.ds(r0, rows)],
                    o_hbm.at[pl.ds(step * 1000 + r0, rows)],
                    sems.at[p],
                )

            @pl.when(i > 0)
            def _():
                mk_copy(i - 1).wait()

            zih = zih_ref[pl.ds(r0, rows), :]
            zil = zil_ref[pl.ds(r0, rows), :]
            scratch[pl.ds(r0, rows)] = (
                lax.dot_general(zih, zjh_ref[...], dn,
                                preferred_element_type=jnp.float32)
                + lax.dot_general(zih, zjl_ref[...], dn,
                                  preferred_element_type=jnp.float32)
                + lax.dot_general(zil, zjh_ref[...], dn,
                                  preferred_element_type=jnp.float32)
            )
            mk_copy(i).start()

        @pl.when(i == n_steps - 1)
        def _():
            for p in range(_DEC_P):
                rows = _DEC_PR if p < _DEC_P - 1 else 1000 - (_DEC_P - 1) * _DEC_PR
                r0 = p * _DEC_PR
                pltpu.make_async_copy(
                    scratch.at[pl.ds(r0, rows)],
                    o_hbm.at[pl.ds(i * 1000 + r0, rows)],
                    sems.at[p],
                ).wait()

    return pl.pallas_call(
        body,
        grid=(n_steps,),
        in_specs=[
            pl.BlockSpec((1000, _H2), lambda i: (i, 0)),
            pl.BlockSpec((1000, _H2), lambda i: (i, 0)),
            pl.BlockSpec((_N, _H2), lambda i: (0, 0)),
            pl.BlockSpec((_N, _H2), lambda i: (0, 0)),
        ],
        out_specs=pl.BlockSpec(memory_space=pl.ANY),
        out_shape=jax.ShapeDtypeStruct((_N, _N), jnp.float32),
        scratch_shapes=[
            pltpu.VMEM((1000, _N), jnp.float32),
            pltpu.SemaphoreType.DMA((_DEC_P,)),
        ],
    )(zh, zl, zh, zl)


def kernel(x, edge_index, edge_weight, W0, W1, W2):
    src = edge_index[0]
    dst = edge_index[1]
    edata = jnp.stack(
        [src.reshape(_NCHUNKS, _CHUNK),
         dst.reshape(_NCHUNKS, _CHUNK)],
        axis=1,
    )
    ew3 = edge_weight.reshape(_NCHUNKS, 1, _CHUNK)
    zeros = jnp.zeros((_N, _H1), jnp.float32)

    h0 = _matmul_xw0(x, W0)
    p = _spmm_partials(h0, edata, ew3, zeros)
    hidden1 = _relu_combine(p)
    q = _spmm_partials(hidden1, edata, ew3, zeros)

    eps = jax.random.normal(jax.random.key(42), (_N, _H2), dtype=jnp.float32)
    z = _z_combine(q, W1, W2, eps)
    zh = z.astype(jnp.bfloat16)
    zl = (z - zh.astype(jnp.float32)).astype(jnp.bfloat16)
    return _decode_manual(zh, zl).reshape(-1)


# speedup vs baseline: 1.1521x; 1.0819x over previous
import functools

import jax
import jax.numpy as jnp
from jax import lax
from jax.experimental import pallas as pl
from jax.experimental.pallas import tpu as pltpu
from jax.experimental.pallas import tpu_sc as plsc

_N = 10000
_E = 160000
_F = 128
_H1 = 32
_H2 = 16

_CHUNK = 128
_NCHUNKS = _E // _CHUNK
_NC = 2
_NS = 16
_NW = _NC * _NS
_CPW = (_NCHUNKS + _NW - 1) // _NW
_NCH_PAD = (_CPW + 2) * _NW
_RPT8 = (_N // _NS) // 8 * 8
_TAIL = _N - _NS * _RPT8

_ROWS_BLK = 1000
_DEC_BM = 512
_DEC_BN = _N


def _spmm_partials(table, edata, ew3, zeros):
    mesh = plsc.VectorSubcoreMesh(core_axis_name="c", subcore_axis_name="s")

    @functools.partial(
        pl.kernel,
        mesh=mesh,
        compiler_params=pltpu.CompilerParams(use_tc_tiling_on_sc=False),
        out_type=jax.ShapeDtypeStruct((_NC, _N, _H1), jnp.float32),
        scratch_types=[
            pltpu.VMEM((2, _CHUNK), jnp.int32),
            pltpu.VMEM((1, _CHUNK), jnp.float32),
            pltpu.VMEM((_CHUNK, _H1), jnp.float32),
            pltpu.VMEM_SHARED((_N, _H1), jnp.float32),
            pltpu.SemaphoreType.DMA,
        ],
    )
    def k(table_hbm, edata_hbm, ew_hbm, zeros_hbm, out_hbm,
          ebuf, wbuf, rows, acc, sem):
        c = lax.axis_index("c")
        s = lax.axis_index("s")
        wid = s * _NC + c

        pltpu.sync_copy(zeros_hbm.at[pl.ds(s * _RPT8, _RPT8)],
                        acc.at[pl.ds(s * _RPT8, _RPT8)])

        @pl.when(s == 0)
        def _():
            pltpu.sync_copy(zeros_hbm.at[pl.ds(_NS * _RPT8, _TAIL)],
                            acc.at[pl.ds(_NS * _RPT8, _TAIL)])

        plsc.subcore_barrier()

        def chunk_body(i, carry):
            chunk = i * _NW + wid

            @pl.when(chunk < _NCHUNKS)
            def _():
                pltpu.sync_copy(edata_hbm.at[chunk], ebuf)
                pltpu.sync_copy(ew_hbm.at[chunk], wbuf)
                pltpu.async_copy(table_hbm.at[ebuf.at[0]], rows, sem).wait()
                for g in range(_CHUNK // 16):
                    wg = wbuf[0, pl.ds(g * 16, 16)]
                    for t in range(16):
                        e = g * 16 + t
                        wb = lax.gather(
                            wg, jnp.full((16, 1), t, jnp.int32),
                            lax.GatherDimensionNumbers(
                                offset_dims=(), collapsed_slice_dims=(0,),
                                start_index_map=(0,)),
                            slice_sizes=(1,),
                            mode=lax.GatherScatterMode.PROMISE_IN_BOUNDS)
                        rows[e, pl.ds(0, 16)] = rows[e, pl.ds(0, 16)] * wb
                        rows[e, pl.ds(16, 16)] = rows[e, pl.ds(16, 16)] * wb
                pltpu.sync_copy(rows, acc.at[ebuf.at[1]], add=True)

            return carry

        lax.fori_loop(0, _CPW, chunk_body, 0)
        plsc.subcore_barrier()
        pltpu.sync_copy(acc.at[pl.ds(s * _RPT8, _RPT8)],
                        out_hbm.at[c, pl.ds(s * _RPT8, _RPT8)])

        @pl.when(s == 0)
        def _():
            pltpu.sync_copy(acc.at[pl.ds(_NS * _RPT8, _TAIL)],
                            out_hbm.at[c, pl.ds(_NS * _RPT8, _TAIL)])

    return k(table, edata, ew3, zeros)


def _matmul_xw0(x, W0):
    def body(x_ref, w_ref, o_ref):
        o_ref[...] = jnp.dot(x_ref[...], w_ref[...],
                             preferred_element_type=jnp.float32)

    return pl.pallas_call(
        body,
        grid=(_N // _ROWS_BLK,),
        in_specs=[
            pl.BlockSpec((_ROWS_BLK, _F), lambda i: (i, 0)),
            pl.BlockSpec((_F, _H1), lambda i: (0, 0)),
        ],
        out_specs=pl.BlockSpec((_ROWS_BLK, _H1), lambda i: (i, 0)),
        out_shape=jax.ShapeDtypeStruct((_N, _H1), jnp.float32),
    )(x, W0)


def _relu_combine(p):
    def body(p_ref, o_ref):
        o_ref[...] = jnp.maximum(p_ref[0] + p_ref[1], 0.0)

    return pl.pallas_call(
        body,
        grid=(_N // _ROWS_BLK,),
        in_specs=[pl.BlockSpec((_NC, _ROWS_BLK, _H1), lambda i: (0, i, 0))],
        out_specs=pl.BlockSpec((_ROWS_BLK, _H1), lambda i: (i, 0)),
        out_shape=jax.ShapeDtypeStruct((_N, _H1), jnp.float32),
    )(p)


def _z_combine(q, W1, W2, eps):
    def body(q_ref, w1_ref, w2_ref, e_ref, o_ref):
        sblk = q_ref[0] + q_ref[1]
        zm = jnp.dot(sblk, w1_ref[...], preferred_element_type=jnp.float32)
        zl = jnp.dot(sblk, w2_ref[...], preferred_element_type=jnp.float32)
        o_ref[...] = zm + e_ref[...] * jnp.exp(zl)

    return pl.pallas_call(
        body,
        grid=(_N // _ROWS_BLK,),
        in_specs=[
            pl.BlockSpec((_NC, _ROWS_BLK, _H1), lambda i: (0, i, 0)),
            pl.BlockSpec((_H1, _H2), lambda i: (0, 0)),
            pl.BlockSpec((_H1, _H2), lambda i: (0, 0)),
            pl.BlockSpec((_ROWS_BLK, _H2), lambda i: (i, 0)),
        ],
        out_specs=pl.BlockSpec((_ROWS_BLK, _H2), lambda i: (i, 0)),
        out_shape=jax.ShapeDtypeStruct((_N, _H2), jnp.float32),
    )(q, W1, W2, eps)


_DEC_P = 8
_DEC_PR = 128


def _decode_manual(z):
    n_steps = _N // 1000
    dn = (((1,), (1,)), ((), ()))

    def body(zi_ref, zj_ref, o_hbm, scratch, sems):
        i = pl.program_id(0)
        for p in range(_DEC_P):
            rows = _DEC_PR if p < _DEC_P - 1 else 1000 - (_DEC_P - 1) * _DEC_PR
            r0 = p * _DEC_PR

            def mk_copy(step):
                return pltpu.make_async_copy(
                    scratch.at[pl.ds(r0, rows)],
                    o_hbm.at[pl.ds(step * 1000 + r0, rows)],
                    sems.at[p],
                )

            @pl.when(i > 0)
            def _():
                mk_copy(i - 1).wait()

            scratch[pl.ds(r0, rows)] = lax.dot_general(
                zi_ref[pl.ds(r0, rows), :], zj_ref[...], dn,
                preferred_element_type=jnp.float32)
            mk_copy(i).start()

        @pl.when(i == n_steps - 1)
        def _():
            for p in range(_DEC_P):
                rows = _DEC_PR if p < _DEC_P - 1 else 1000 - (_DEC_P - 1) * _DEC_PR
                r0 = p * _DEC_PR
                pltpu.make_async_copy(
                    scratch.at[pl.ds(r0, rows)],
                    o_hbm.at[pl.ds(i * 1000 + r0, rows)],
                    sems.at[p],
                ).wait()

    return pl.pallas_call(
        body,
        grid=(n_steps,),
        in_specs=[
            pl.BlockSpec((1000, _H2), lambda i: (i, 0)),
            pl.BlockSpec((_N, _H2), lambda i: (0, 0)),
        ],
        out_specs=pl.BlockSpec(memory_space=pl.ANY),
        out_shape=jax.ShapeDtypeStruct((_N, _N), jnp.float32),
        scratch_shapes=[
            pltpu.VMEM((1000, _N), jnp.float32),
            pltpu.SemaphoreType.DMA((_DEC_P,)),
        ],
    )(z, z)


def kernel(x, edge_index, edge_weight, W0, W1, W2):
    src = edge_index[0]
    dst = edge_index[1]
    edata = jnp.stack(
        [src.reshape(_NCHUNKS, _CHUNK),
         dst.reshape(_NCHUNKS, _CHUNK)],
        axis=1,
    )
    ew3 = edge_weight.reshape(_NCHUNKS, 1, _CHUNK)
    zeros = jnp.zeros((_N, _H1), jnp.float32)

    h0 = _matmul_xw0(x, W0)
    p = _spmm_partials(h0, edata, ew3, zeros)
    hidden1 = _relu_combine(p)
    q = _spmm_partials(hidden1, edata, ew3, zeros)

    eps = jax.random.normal(jax.random.key(42), (_N, _H2), dtype=jnp.float32)
    z = _z_combine(q, W1, W2, eps)
    return _decode_manual(z).reshape(-1)
